# baseline (speedup 1.0000x reference)
import functools

import jax
import jax.numpy as jnp
from jax import lax
from jax.experimental import pallas as pl
from jax.experimental.pallas import tpu as pltpu
from jax.experimental.pallas import tpu_sc as plsc

N = 10000
E = 320000
H = 16
F_IN = 128

NC = 2
NS = 16
NW = NC * NS
EPT = E // NW
CHE = 80
NCHE = EPT // CHE
KB = 10
NG = NCHE // (2 * KB)
TAIL = NCHE - NG * 2 * KB
KD = 25
KE = 25
NPAD = 10240
RPT = NPAD // NS
GRS = N // NS
N8 = N // 8
NP8 = NPAD // 8

_mesh = plsc.VectorSubcoreMesh(core_axis_name="c", subcore_axis_name="s")


NCOL = E // 128
NBLK = NCOL // 8
NB2 = NBLK // NW
XBLK = NBLK - NB2 * NW
PBASE = NBLK * 1024


@functools.partial(
    pl.kernel,
    out_type=[jax.ShapeDtypeStruct((NC * NPAD * H,), jnp.float32),
              jax.ShapeDtypeStruct((E,), jnp.int32),
              jax.ShapeDtypeStruct((E,), jnp.int32)],
    mesh=_mesh,
    scratch_types=[
        pltpu.VMEM((NB2, 2, 1024), jnp.int32),
        pltpu.VMEM((2, 1024), jnp.int32),
        pltpu.VMEM((128,), jnp.float32),
        pltpu.VMEM((RPT,), jnp.float32),
        pltpu.VMEM((RPT * H,), jnp.float32),
        pltpu.VMEM_SHARED((NPAD,), jnp.float32),
        pltpu.SemaphoreType.DMA,
        pltpu.SemaphoreType.DMA,
        pltpu.SemaphoreType.DMA,
        pltpu.SemaphoreType.DMA,
    ],
)
def _deg_kernel(ei_hbm, out_hbm, osrc_hbm, odst_hbm,
                buf_v, xbuf_v, ones_v, accv_v, exp_v, acc_sh,
                sin, sdn, sout, shist):
    c = lax.axis_index("c")
    s = lax.axis_index("s")
    wid = s * NC + c
    blk0 = wid * NB2
    for k in range(128 // 16):
        ones_v[pl.ds(16 * k, 16)] = jnp.ones((16,), jnp.float32)
    for k in range(RPT // 16):
        accv_v[pl.ds(16 * k, 16)] = jnp.zeros((16,), jnp.float32)
    pltpu.sync_copy(accv_v, acc_sh.at[pl.ds(s * RPT, RPT)])

    cf = [pltpu.async_copy(
              ei_hbm.at[pl.ds(0, 2), pl.ds(1024 * (blk0 + b), 1024)],
              buf_v.at[b], sin)
          for b in range(NB2)]
    couts = []
    for b in range(NB2):
        cf[b].wait()
        couts.append(pltpu.async_copy(
            buf_v.at[b, 0], osrc_hbm.at[pl.ds(1024 * (blk0 + b), 1024)],
            sout))
        couts.append(pltpu.async_copy(
            buf_v.at[b, 1], odst_hbm.at[pl.ds(1024 * (blk0 + b), 1024)],
            sout))

    @pl.when(wid < XBLK)
    def _():
        xc = NW * NB2 + wid
        pltpu.async_copy(ei_hbm.at[pl.ds(0, 2), pl.ds(1024 * xc, 1024)],
                         xbuf_v, sin).wait()
        pltpu.async_copy(xbuf_v.at[0],
                         osrc_hbm.at[pl.ds(1024 * xc, 1024)], sout).wait()
        pltpu.async_copy(xbuf_v.at[1],
                         odst_hbm.at[pl.ds(1024 * xc, 1024)], sout).wait()

    @pl.when(wid == XBLK)
    def _():
        pltpu.async_copy(ei_hbm.at[pl.ds(0, 2), pl.ds(PBASE, 512)],
                         xbuf_v.at[pl.ds(0, 2), pl.ds(0, 512)], sin).wait()
        pltpu.async_copy(xbuf_v.at[0, pl.ds(0, 512)],
                         osrc_hbm.at[pl.ds(PBASE, 512)], sout).wait()
        pltpu.async_copy(xbuf_v.at[1, pl.ds(0, 512)],
                         odst_hbm.at[pl.ds(PBASE, 512)], sout).wait()

    plsc.subcore_barrier()

    ch = [pltpu.async_copy(ones_v,
                           acc_sh.at[buf_v.at[b, 1, pl.ds(128 * j, 128)]],
                           shist, add=True)
          for b in range(NB2) for j in range(8)]
    for cp in ch:
        cp.wait()

    @pl.when(wid < XBLK)
    def _():
        chx = [pltpu.async_copy(ones_v,
                                acc_sh.at[xbuf_v.at[1, pl.ds(128 * j, 128)]],
                                shist, add=True)
               for j in range(8)]
        for cp in chx:
            cp.wait()

    @pl.when(wid == XBLK)
    def _():
        chp = [pltpu.async_copy(ones_v,
                                acc_sh.at[xbuf_v.at[1, pl.ds(128 * j, 128)]],
                                shist, add=True)
               for j in range(4)]
        for cp in chp:
            cp.wait()

    for cp in couts:
        cp.wait()
    plsc.subcore_barrier()
    pltpu.sync_copy(acc_sh.at[pl.ds(s * RPT, RPT)], accv_v)

    def expand(i, carry):
        v = accv_v[pl.ds(16 * i, 16)]
        for j in range(16):
            exp_v[pl.ds(H * (16 * i + j), H)] = jnp.broadcast_to(v[j], (H,))
        return carry

    lax.fori_loop(0, RPT // 16, expand, 0)
    pltpu.sync_copy(exp_v,
                    out_hbm.at[pl.ds((c * NPAD + s * RPT) * H, RPT * H)])


@functools.partial(
    pl.kernel,
    out_type=jax.ShapeDtypeStruct((NC * NPAD, H), jnp.float32),
    mesh=_mesh,
    compiler_params=pltpu.CompilerParams(use_tc_tiling_on_sc=False),
    scratch_types=[
        pltpu.VMEM((NCHE, CHE), jnp.int32),
        pltpu.VMEM((NCHE, CHE), jnp.int32),
        pltpu.VMEM((KB, CHE, H), jnp.float32),
        pltpu.VMEM((KB, CHE, H), jnp.float32),
        pltpu.VMEM((RPT, H), jnp.float32),
        pltpu.VMEM((GRS, H), jnp.float32),
        pltpu.VMEM_SHARED((NPAD, H), jnp.float32),
        pltpu.VMEM_SHARED((N, H), jnp.float32),
        pltpu.SemaphoreType.DMA,
        pltpu.SemaphoreType.DMA,
        pltpu.SemaphoreType.DMA,
        pltpu.SemaphoreType.DMA,
    ],
)
def _scat_kernel(g_hbm, src_hbm, dst_hbm, z_hbm, out_hbm,
                 src_v, dst_v, ra_v, rb_v, zb_v, gb_v, acc_sh, g_sh,
                 sga, sgb, ssa, ssb):
    c = lax.axis_index("c")
    s = lax.axis_index("s")
    wid = s * NC + c
    cp1 = pltpu.async_copy(src_hbm.at[wid], src_v, sga)
    cp2 = pltpu.async_copy(dst_hbm.at[wid], dst_v, sgb)
    cp3 = pltpu.async_copy(g_hbm.at[pl.ds(s * GRS, GRS)], gb_v, ssa)
    cp4 = pltpu.async_copy(z_hbm, zb_v, ssb)
    cp3.wait()
    cp5 = pltpu.async_copy(gb_v, g_sh.at[pl.ds(s * GRS, GRS)], ssa)
    cp4.wait()
    cp6 = pltpu.async_copy(zb_v, acc_sh.at[pl.ds(s * RPT, RPT)], ssb)
    cp1.wait()
    cp2.wait()
    cp5.wait()
    cp6.wait()
    plsc.subcore_barrier()

    def body(g, carry):
        j0 = g * (2 * KB)
        cga = [pltpu.async_copy(g_sh.at[src_v.at[j0 + b]], ra_v.at[b], sga)
               for b in range(KB)]
        cgb = [pltpu.async_copy(g_sh.at[src_v.at[j0 + KB + b]], rb_v.at[b],
                                sgb)
               for b in range(KB)]
        for cp in cga:
            cp.wait()
        csa = [pltpu.async_copy(ra_v.at[b], acc_sh.at[dst_v.at[j0 + b]],
                                ssa, add=True)
               for b in range(KB)]
        for cp in cgb:
            cp.wait()
        csb = [pltpu.async_copy(rb_v.at[b], acc_sh.at[dst_v.at[j0 + KB + b]],
                                ssb, add=True)
               for b in range(KB)]
        for cp in csa:
            cp.wait()
        for cp in csb:
            cp.wait()
        return carry

    lax.fori_loop(0, NG, body, 0)

    t0 = NG * 2 * KB
    ct = [pltpu.async_copy(g_sh.at[src_v.at[t0 + b]], ra_v.at[b], sga)
          for b in range(TAIL)]
    for cp in ct:
        cp.wait()
    cs = [pltpu.async_copy(ra_v.at[b], acc_sh.at[dst_v.at[t0 + b]],
                           ssa, add=True)
          for b in range(TAIL)]
    for cp in cs:
        cp.wait()

    plsc.subcore_barrier()
    pltpu.sync_copy(acc_sh.at[pl.ds(s * RPT, RPT)],
                    out_hbm.at[pl.ds(c * NPAD + s * RPT, RPT)])


@functools.partial(
    pl.kernel,
    out_type=jax.ShapeDtypeStruct((E,), jnp.float32),
    mesh=_mesh,
    compiler_params=pltpu.CompilerParams(use_tc_tiling_on_sc=False,
                                         needs_layout_passes=False),
    scratch_types=[
        pltpu.VMEM((N,), jnp.float32),
        pltpu.VMEM((N,), jnp.float32),
        pltpu.VMEM((EPT,), jnp.int32),
        pltpu.VMEM((EPT,), jnp.int32),
        pltpu.VMEM((EPT,), jnp.float32),
        pltpu.SemaphoreType.DMA,
        pltpu.SemaphoreType.DMA,
    ],
)
def _edge_kernel(a_hbm, c_hbm, src_hbm, dst_hbm, out_hbm,
                 a_v, c_v, src_v, dst_v, vo_v, sm1, sm2):
    c = lax.axis_index("c")
    s = lax.axis_index("s")
    wid = s * NC + c
    cps = [pltpu.async_copy(a_hbm, a_v, sm1),
           pltpu.async_copy(c_hbm, c_v, sm2),
           pltpu.async_copy(src_hbm.at[wid], src_v, sm1),
           pltpu.async_copy(dst_hbm.at[wid], dst_v, sm2)]
    for cp in cps:
        cp.wait()

    def body(i, carry):
        for u in range(5):
            sl = pl.ds(80 * i + 16 * u, 16)
            av = plsc.load_gather(a_v, [src_v[sl]])
            cv = plsc.load_gather(c_v, [dst_v[sl]])
            vo_v[sl] = av + cv
        return carry

    lax.fori_loop(0, EPT // 80, body, 0)
    pltpu.sync_copy(vo_v, out_hbm.at[pl.ds(wid * EPT, EPT)])


def _tc_h0_body(x3_ref, w1_ref, h0_ref):
    h0_ref[...] = jnp.concatenate(
        [jnp.dot(x3_ref[:, k, :], w1_ref[...],
                 preferred_element_type=jnp.float32)
         for k in range(8)], axis=1)


_tc_h0 = pl.pallas_call(
    _tc_h0_body,
    out_shape=jax.ShapeDtypeStruct((N8, 128), jnp.float32),
)


def _tc_a_body(degp_ref, h0_ref, b1p_ref,
               g0_ref, self1_ref, dis_ref, invd_ref):
    deg = degp_ref[0, :N8, :] + degp_ref[1, :N8, :] + 1.0
    dis_p = lax.rsqrt(deg)
    invd_p = 1.0 / deg
    dis_ref[...] = dis_p
    invd_ref[...] = invd_p
    h0p = h0_ref[...]
    g0_ref[...] = h0p * dis_p
    self1_ref[...] = h0p * invd_p + b1p_ref[...]


_tc_a = pl.pallas_call(
    _tc_a_body,
    out_shape=[jax.ShapeDtypeStruct((N8, 128), jnp.float32),
               jax.ShapeDtypeStruct((N8, 128), jnp.float32),
               jax.ShapeDtypeStruct((N8, 128), jnp.float32),
               jax.ShapeDtypeStruct((N8, 128), jnp.float32)],
)


def _tc_b_body(s1_ref, self1_ref, dis_ref, invd_ref, w2b_ref, b2p_ref,
               g1_ref, self2_ref):
    ssum = s1_ref[0, :N8, :] + s1_ref[1, :N8, :]
    h1p = jnp.maximum(dis_ref[...] * ssum + self1_ref[...], 0.0)
    h1wp = jnp.dot(h1p, w2b_ref[...], preferred_element_type=jnp.float32)
    g1_ref[...] = h1wp * dis_ref[...]
    self2_ref[...] = h1wp * invd_ref[...] + b2p_ref[...]


_tc_b = pl.pallas_call(
    _tc_b_body,
    out_shape=[jax.ShapeDtypeStruct((N8, 128), jnp.float32),
               jax.ShapeDtypeStruct((N8, 128), jnp.float32)],
)


def _tc_c_body(s2_ref, self2_ref, dis_ref, wfab_ref, bfv_ref, ac_ref):
    ssum = s2_ref[0, :N8, :] + s2_ref[1, :N8, :]
    h2p = dis_ref[...] * ssum + self2_ref[...]
    ac_ref[...] = jnp.dot(h2p, wfab_ref[...],
                          preferred_element_type=jnp.float32) + bfv_ref[...]


_tc_c = pl.pallas_call(
    _tc_c_body,
    out_shape=jax.ShapeDtypeStruct((N8, 16), jnp.float32),
)


def kernel(x, edge_index, W1, b1, W2, b2, Wf, bf):
    z16 = jnp.zeros((RPT, H), jnp.float32)
    x3 = x.reshape(N8, 8, F_IN)
    eye8 = jnp.eye(8, dtype=jnp.float32)
    w2b = jnp.kron(eye8, W2)
    wfab = jnp.concatenate([jnp.kron(eye8, Wf[:H]),
                            jnp.kron(eye8, Wf[H:])], axis=1)
    b1p = jnp.tile(b1, 8)
    b2p = jnp.tile(b2, 8)
    bfv = jnp.concatenate([jnp.broadcast_to(bf, (8,)),
                           jnp.zeros((8,), jnp.float32)])

    h0p = _tc_h0(x3, W1)
    degp_flat, osrc, odst = _deg_kernel(edge_index)
    degp = degp_flat.reshape(NC, NP8, 128)
    src3 = osrc.reshape(NW, NCHE, CHE)
    dst3 = odst.reshape(NW, NCHE, CHE)
    g0p, self1p, disp, invdp = _tc_a(degp, h0p, b1p)
    s1 = _scat_kernel(g0p.reshape(N, H), src3, dst3, z16).reshape(NC, NP8, 128)
    g1p, self2p = _tc_b(s1, self1p, disp, invdp, w2b, b2p)
    s2 = _scat_kernel(g1p.reshape(N, H), src3, dst3, z16).reshape(NC, NP8, 128)
    ac = _tc_c(s2, self2p, disp, wfab, bfv)
    a2 = ac[:, 0:8].reshape(N)
    c2 = ac[:, 8:16].reshape(N)
    src2 = src3.reshape(NW, EPT)
    dst2 = dst3.reshape(NW, EPT)
    return _edge_kernel(a2, c2, src2, dst2)


# speedup vs baseline: 1.0161x; 1.0161x over previous
import functools

import jax
import jax.numpy as jnp
from jax import lax
from jax.experimental import pallas as pl
from jax.experimental.pallas import tpu as pltpu
from jax.experimental.pallas import tpu_sc as plsc

N = 10000
E = 320000
H = 16
F_IN = 128

NC = 2
NS = 16
NW = NC * NS
EPT = E // NW
CHE = 80
NCHE = EPT // CHE
KB = 10
NG = NCHE // (2 * KB)
TAIL = NCHE - NG * 2 * KB
KD = 25
KE = 25
NPAD = 10240
RPT = NPAD // NS
GRS = N // NS
N8 = N // 8
NP8 = NPAD // 8

_mesh = plsc.VectorSubcoreMesh(core_axis_name="c", subcore_axis_name="s")


NCOL = E // 128
NBLK = NCOL // 8
NB2 = NBLK // NW
XBLK = NBLK - NB2 * NW
PBASE = NBLK * 1024


@functools.partial(
    pl.kernel,
    out_type=[jax.ShapeDtypeStruct((NC * NPAD * H,), jnp.float32),
              jax.ShapeDtypeStruct((E,), jnp.int32),
              jax.ShapeDtypeStruct((E,), jnp.int32)],
    mesh=_mesh,
    scratch_types=[
        pltpu.VMEM((NB2, 2, 1024), jnp.int32),
        pltpu.VMEM((2, 1024), jnp.int32),
        pltpu.VMEM((128,), jnp.float32),
        pltpu.VMEM((RPT,), jnp.float32),
        pltpu.VMEM((RPT * H,), jnp.float32),
        pltpu.VMEM_SHARED((NPAD,), jnp.float32),
        pltpu.SemaphoreType.DMA,
        pltpu.SemaphoreType.DMA,
        pltpu.SemaphoreType.DMA,
        pltpu.SemaphoreType.DMA,
    ],
)
def _deg_kernel(ei_hbm, out_hbm, osrc_hbm, odst_hbm,
                buf_v, xbuf_v, ones_v, accv_v, exp_v, acc_sh,
                sin, sdn, sout, shist):
    c = lax.axis_index("c")
    s = lax.axis_index("s")
    wid = s * NC + c
    blk0 = wid * NB2
    for k in range(128 // 16):
        ones_v[pl.ds(16 * k, 16)] = jnp.ones((16,), jnp.float32)
    for k in range(RPT // 16):
        accv_v[pl.ds(16 * k, 16)] = jnp.zeros((16,), jnp.float32)
    pltpu.sync_copy(accv_v, acc_sh.at[pl.ds(s * RPT, RPT)])

    cf = [pltpu.async_copy(
              ei_hbm.at[pl.ds(0, 2), pl.ds(1024 * (blk0 + b), 1024)],
              buf_v.at[b], sin)
          for b in range(NB2)]
    couts = []
    for b in range(NB2):
        cf[b].wait()
        couts.append(pltpu.async_copy(
            buf_v.at[b, 0], osrc_hbm.at[pl.ds(1024 * (blk0 + b), 1024)],
            sout))
        couts.append(pltpu.async_copy(
            buf_v.at[b, 1], odst_hbm.at[pl.ds(1024 * (blk0 + b), 1024)],
            sout))

    @pl.when(wid < XBLK)
    def _():
        xc = NW * NB2 + wid
        pltpu.async_copy(ei_hbm.at[pl.ds(0, 2), pl.ds(1024 * xc, 1024)],
                         xbuf_v, sin).wait()
        pltpu.async_copy(xbuf_v.at[0],
                         osrc_hbm.at[pl.ds(1024 * xc, 1024)], sout).wait()
        pltpu.async_copy(xbuf_v.at[1],
                         odst_hbm.at[pl.ds(1024 * xc, 1024)], sout).wait()

    @pl.when(wid == XBLK)
    def _():
        pltpu.async_copy(ei_hbm.at[pl.ds(0, 2), pl.ds(PBASE, 512)],
                         xbuf_v.at[pl.ds(0, 2), pl.ds(0, 512)], sin).wait()
        pltpu.async_copy(xbuf_v.at[0, pl.ds(0, 512)],
                         osrc_hbm.at[pl.ds(PBASE, 512)], sout).wait()
        pltpu.async_copy(xbuf_v.at[1, pl.ds(0, 512)],
                         odst_hbm.at[pl.ds(PBASE, 512)], sout).wait()

    plsc.subcore_barrier()

    ch = [pltpu.async_copy(ones_v,
                           acc_sh.at[buf_v.at[b, 1, pl.ds(128 * j, 128)]],
                           shist, add=True)
          for b in range(NB2) for j in range(8)]
    for cp in ch:
        cp.wait()

    @pl.when(wid < XBLK)
    def _():
        chx = [pltpu.async_copy(ones_v,
                                acc_sh.at[xbuf_v.at[1, pl.ds(128 * j, 128)]],
                                shist, add=True)
               for j in range(8)]
        for cp in chx:
            cp.wait()

    @pl.when(wid == XBLK)
    def _():
        chp = [pltpu.async_copy(ones_v,
                                acc_sh.at[xbuf_v.at[1, pl.ds(128 * j, 128)]],
                                shist, add=True)
               for j in range(4)]
        for cp in chp:
            cp.wait()

    for cp in couts:
        cp.wait()
    plsc.subcore_barrier()
    pltpu.sync_copy(acc_sh.at[pl.ds(s * RPT, RPT)], accv_v)

    def expand(i, carry):
        v = accv_v[pl.ds(16 * i, 16)]
        for j in range(16):
            exp_v[pl.ds(H * (16 * i + j), H)] = jnp.broadcast_to(v[j], (H,))
        return carry

    lax.fori_loop(0, RPT // 16, expand, 0)
    pltpu.sync_copy(exp_v,
                    out_hbm.at[pl.ds((c * NPAD + s * RPT) * H, RPT * H)])


@functools.partial(
    pl.kernel,
    out_type=jax.ShapeDtypeStruct((NC * NPAD, H), jnp.float32),
    mesh=_mesh,
    compiler_params=pltpu.CompilerParams(use_tc_tiling_on_sc=False),
    scratch_types=[
        pltpu.VMEM((NCHE, CHE), jnp.int32),
        pltpu.VMEM((NCHE, CHE), jnp.int32),
        pltpu.VMEM((KB, CHE, H), jnp.float32),
        pltpu.VMEM((KB, CHE, H), jnp.float32),
        pltpu.VMEM((RPT, H), jnp.float32),
        pltpu.VMEM((GRS, H), jnp.float32),
        pltpu.VMEM_SHARED((NPAD, H), jnp.float32),
        pltpu.VMEM_SHARED((N, H), jnp.float32),
        pltpu.SemaphoreType.DMA,
        pltpu.SemaphoreType.DMA,
        pltpu.SemaphoreType.DMA,
        pltpu.SemaphoreType.DMA,
    ],
)
def _scat_kernel(g_hbm, src_hbm, dst_hbm, z_hbm, out_hbm,
                 src_v, dst_v, ra_v, rb_v, zb_v, gb_v, acc_sh, g_sh,
                 sga, sgb, ssa, ssb):
    c = lax.axis_index("c")
    s = lax.axis_index("s")
    wid = s * NC + c
    cp1 = pltpu.async_copy(src_hbm.at[wid], src_v, sga)
    cp2 = pltpu.async_copy(dst_hbm.at[wid], dst_v, sgb)
    cp3 = pltpu.async_copy(g_hbm.at[pl.ds(s * GRS, GRS)], gb_v, ssa)
    cp4 = pltpu.async_copy(z_hbm.at[pl.ds(s * RPT, RPT)], zb_v, ssb)
    cp3.wait()
    cp5 = pltpu.async_copy(gb_v, g_sh.at[pl.ds(s * GRS, GRS)], ssa)
    cp4.wait()
    cp6 = pltpu.async_copy(zb_v, acc_sh.at[pl.ds(s * RPT, RPT)], ssb)
    cp1.wait()
    cp2.wait()
    cp5.wait()
    cp6.wait()
    plsc.subcore_barrier()

    def body(g, carry):
        j0 = g * (2 * KB)
        cga = [pltpu.async_copy(g_sh.at[src_v.at[j0 + b]], ra_v.at[b], sga)
               for b in range(KB)]
        cgb = [pltpu.async_copy(g_sh.at[src_v.at[j0 + KB + b]], rb_v.at[b],
                                sgb)
               for b in range(KB)]
        for cp in cga:
            cp.wait()
        csa = [pltpu.async_copy(ra_v.at[b], acc_sh.at[dst_v.at[j0 + b]],
                                ssa, add=True)
               for b in range(KB)]
        for cp in cgb:
            cp.wait()
        csb = [pltpu.async_copy(rb_v.at[b], acc_sh.at[dst_v.at[j0 + KB + b]],
                                ssb, add=True)
               for b in range(KB)]
        for cp in csa:
            cp.wait()
        for cp in csb:
            cp.wait()
        return carry

    lax.fori_loop(0, NG, body, 0)

    t0 = NG * 2 * KB
    ct = [pltpu.async_copy(g_sh.at[src_v.at[t0 + b]], ra_v.at[b], sga)
          for b in range(TAIL)]
    for cp in ct:
        cp.wait()
    cs = [pltpu.async_copy(ra_v.at[b], acc_sh.at[dst_v.at[t0 + b]],
                           ssa, add=True)
          for b in range(TAIL)]
    for cp in cs:
        cp.wait()

    plsc.subcore_barrier()
    pltpu.sync_copy(acc_sh.at[pl.ds(s * RPT, RPT)],
                    out_hbm.at[pl.ds(c * NPAD + s * RPT, RPT)])


@functools.partial(
    pl.kernel,
    out_type=jax.ShapeDtypeStruct((E,), jnp.float32),
    mesh=_mesh,
    compiler_params=pltpu.CompilerParams(use_tc_tiling_on_sc=False,
                                         needs_layout_passes=False),
    scratch_types=[
        pltpu.VMEM((N,), jnp.float32),
        pltpu.VMEM((N,), jnp.float32),
        pltpu.VMEM((EPT,), jnp.int32),
        pltpu.VMEM((EPT,), jnp.int32),
        pltpu.VMEM((EPT,), jnp.float32),
    ],
)
def _edge_kernel(a_hbm, c_hbm, src_hbm, dst_hbm, out_hbm,
                 a_v, c_v, src_v, dst_v, vo_v):
    c = lax.axis_index("c")
    s = lax.axis_index("s")
    wid = s * NC + c
    pltpu.sync_copy(a_hbm, a_v)
    pltpu.sync_copy(c_hbm, c_v)
    pltpu.sync_copy(src_hbm.at[wid], src_v)
    pltpu.sync_copy(dst_hbm.at[wid], dst_v)

    def body(i, carry):
        for u in range(5):
            sl = pl.ds(80 * i + 16 * u, 16)
            av = plsc.load_gather(a_v, [src_v[sl]])
            cv = plsc.load_gather(c_v, [dst_v[sl]])
            vo_v[sl] = av + cv
        return carry

    lax.fori_loop(0, EPT // 80, body, 0)
    pltpu.sync_copy(vo_v, out_hbm.at[pl.ds(wid * EPT, EPT)])


def _tc_h0_body(x3_ref, w1_ref, h0_ref):
    h0_ref[...] = jnp.concatenate(
        [jnp.dot(x3_ref[:, k, :], w1_ref[...],
                 preferred_element_type=jnp.float32)
         for k in range(8)], axis=1)


_tc_h0 = pl.pallas_call(
    _tc_h0_body,
    out_shape=jax.ShapeDtypeStruct((N8, 128), jnp.float32),
)


def _tc_a_body(degp_ref, h0_ref, b1p_ref,
               g0_ref, self1_ref, dis_ref, invd_ref):
    deg = degp_ref[0, :N8, :] + degp_ref[1, :N8, :] + 1.0
    dis_p = lax.rsqrt(deg)
    invd_p = 1.0 / deg
    dis_ref[...] = dis_p
    invd_ref[...] = invd_p
    h0p = h0_ref[...]
    g0_ref[...] = h0p * dis_p
    self1_ref[...] = h0p * invd_p + b1p_ref[...]


_tc_a = pl.pallas_call(
    _tc_a_body,
    out_shape=[jax.ShapeDtypeStruct((N8, 128), jnp.float32),
               jax.ShapeDtypeStruct((N8, 128), jnp.float32),
               jax.ShapeDtypeStruct((N8, 128), jnp.float32),
               jax.ShapeDtypeStruct((N8, 128), jnp.float32)],
)


def _tc_b_body(s1_ref, self1_ref, dis_ref, invd_ref, w2b_ref, b2p_ref,
               g1_ref, self2_ref):
    ssum = s1_ref[0, :N8, :] + s1_ref[1, :N8, :]
    h1p = jnp.maximum(dis_ref[...] * ssum + self1_ref[...], 0.0)
    h1wp = jnp.dot(h1p, w2b_ref[...], preferred_element_type=jnp.float32)
    g1_ref[...] = h1wp * dis_ref[...]
    self2_ref[...] = h1wp * invd_ref[...] + b2p_ref[...]


_tc_b = pl.pallas_call(
    _tc_b_body,
    out_shape=[jax.ShapeDtypeStruct((N8, 128), jnp.float32),
               jax.ShapeDtypeStruct((N8, 128), jnp.float32)],
)


def _tc_c_body(s2_ref, self2_ref, dis_ref, wfab_ref, bfv_ref, ac_ref):
    ssum = s2_ref[0, :N8, :] + s2_ref[1, :N8, :]
    h2p = dis_ref[...] * ssum + self2_ref[...]
    ac_ref[...] = jnp.dot(h2p, wfab_ref[...],
                          preferred_element_type=jnp.float32) + bfv_ref[...]


_tc_c = pl.pallas_call(
    _tc_c_body,
    out_shape=jax.ShapeDtypeStruct((N8, 16), jnp.float32),
)


def kernel(x, edge_index, W1, b1, W2, b2, Wf, bf):
    z16 = jnp.zeros((NPAD, H), jnp.float32)
    x3 = x.reshape(N8, 8, F_IN)
    eye8 = jnp.eye(8, dtype=jnp.float32)
    w2b = jnp.kron(eye8, W2)
    wfab = jnp.concatenate([jnp.kron(eye8, Wf[:H]),
                            jnp.kron(eye8, Wf[H:])], axis=1)
    b1p = jnp.tile(b1, 8)
    b2p = jnp.tile(b2, 8)
    bfv = jnp.concatenate([jnp.broadcast_to(bf, (8,)),
                           jnp.zeros((8,), jnp.float32)])

    h0p = _tc_h0(x3, W1)
    degp_flat, osrc, odst = _deg_kernel(edge_index)
    degp = degp_flat.reshape(NC, NP8, 128)
    src3 = osrc.reshape(NW, NCHE, CHE)
    dst3 = odst.reshape(NW, NCHE, CHE)
    g0p, self1p, disp, invdp = _tc_a(degp, h0p, b1p)
    s1 = _scat_kernel(g0p.reshape(N, H), src3, dst3, z16).reshape(NC, NP8, 128)
    g1p, self2p = _tc_b(s1, self1p, disp, invdp, w2b, b2p)
    s2 = _scat_kernel(g1p.reshape(N, H), src3, dst3, z16).reshape(NC, NP8, 128)
    ac = _tc_c(s2, self2p, disp, wfab, bfv)
    a2 = ac[:, 0:8].reshape(N)
    c2 = ac[:, 8:16].reshape(N)
    src2 = src3.reshape(NW, EPT)
    dst2 = dst3.reshape(NW, EPT)
    return _edge_kernel(a2, c2, src2, dst2)
